# baseline (speedup 1.0000x reference)
import jax
import jax.numpy as jnp
from jax import lax
from jax.experimental import pallas as pl
from jax.experimental.pallas import tpu as pltpu
from jax.experimental.pallas import tpu_sc as plsc

N_SRC = 10000
N_DST = 10000
E = 640000
D = 128

NC = 2
NS = 16
NP = 4
C = D // NP
CHUNK = 256
BLK = 8
NBLK = 20
CPT = BLK * NBLK
E_PAD = NS * CPT * CHUNK
ACC_ROWS = 10016
ROWS_PT = ACC_ROWS // NS
DUMMY = 10008


def _make_agg(with_deg: bool):
    mesh = plsc.VectorSubcoreMesh(core_axis_name="c", subcore_axis_name="s")

    out_type = [
        pltpu.HBM((NP, ACC_ROWS, C), jnp.float32),
        pltpu.HBM((NP, ACC_ROWS, C), jnp.float32),
    ]
    if with_deg:
        out_type += [
            pltpu.HBM((ACC_ROWS, 16), jnp.float32),
            pltpu.HBM((ACC_ROWS, 16), jnp.float32),
        ]

    scratch = [
        pltpu.VMEM((BLK, CHUNK), jnp.int32),
        pltpu.VMEM((BLK, CHUNK), jnp.int32),
        pltpu.VMEM((CHUNK, C), jnp.float32),
        pltpu.VMEM((CHUNK, C), jnp.float32),
        pltpu.SemaphoreType.DMA,
        pltpu.SemaphoreType.DMA,
        pltpu.SemaphoreType.DMA,
        pltpu.SemaphoreType.DMA,
        pltpu.VMEM_SHARED((ACC_ROWS, C), jnp.float32),
    ]
    if with_deg:
        scratch += [
            pltpu.VMEM((CHUNK, 16), jnp.float32),
            pltpu.VMEM_SHARED((ACC_ROWS, 16), jnp.float32),
        ]

    def body(tab_f, tab_r, gidx_f, sidx_f, gidx_r, sidx_r, zeros_hbm,
             zeros16_hbm, ones_hbm, *rest):
        if with_deg:
            (out_f, out_r, dout_f, dout_r,
             gidx_v, sidx_v, rows_a, rows_b, gsem_a, gsem_b, ssem_a, ssem_b,
             acc, ones_v, dacc) = rest
        else:
            (out_f, out_r, gidx_v, sidx_v, rows_a, rows_b, gsem_a, gsem_b,
             ssem_a, ssem_b, acc) = rest
            dout_f = dout_r = ones_v = dacc = None

        cid = lax.axis_index("c")
        sid = lax.axis_index("s")
        base = sid * ROWS_PT

        def run(tab, gidx_hbm, sidx_hbm, out_hbm, dout_hbm):
            if with_deg:
                pltpu.sync_copy(ones_hbm, ones_v)
                pltpu.sync_copy(zeros16_hbm, dacc.at[pl.ds(base, ROWS_PT)])

            for p in range(NP):
                pltpu.sync_copy(zeros_hbm, acc.at[pl.ds(base, ROWS_PT)])
                plsc.subcore_barrier()

                deg_here = with_deg and p == 0

                def blk(b, carry):
                    pltpu.sync_copy(gidx_hbm.at[p, sid, pl.ds(b * BLK, BLK)],
                                    gidx_v)
                    pltpu.sync_copy(sidx_hbm.at[sid, pl.ds(b * BLK, BLK)],
                                    sidx_v)

                    bufs = (rows_a, rows_b)
                    gsems = (gsem_a, gsem_b)
                    ssems = (ssem_a, ssem_b)

                    def gather(j, x):
                        return pltpu.async_copy(tab.at[gidx_v.at[j]],
                                                bufs[x], gsems[x])

                    gh = [gather(0, 0), None]
                    sh = [None, None]
                    for j in range(BLK):
                        x = j % 2
                        y = 1 - x
                        if j + 1 < BLK:
                            if sh[y] is not None:
                                sh[y].wait()
                            gh[y] = gather(j + 1, y)
                        gh[x].wait()
                        sh[x] = pltpu.async_copy(
                            bufs[x], acc.at[sidx_v.at[j]], ssems[x],
                            add=True)
                        if deg_here:
                            pltpu.sync_copy(ones_v, dacc.at[sidx_v.at[j]],
                                            add=True)
                    sh[0].wait()
                    sh[1].wait()
                    return carry

                lax.fori_loop(0, NBLK, blk, 0)
                plsc.subcore_barrier()

                pltpu.sync_copy(acc.at[pl.ds(base, ROWS_PT)],
                                out_hbm.at[p, pl.ds(base, ROWS_PT)])
                if with_deg and p == 0:
                    pltpu.sync_copy(dacc.at[pl.ds(base, ROWS_PT)],
                                    dout_hbm.at[pl.ds(base, ROWS_PT)])
                plsc.subcore_barrier()

        @pl.when(cid == 0)
        def _fwd():
            run(tab_f, gidx_f, sidx_f, out_f, dout_f)

        @pl.when(cid == 1)
        def _rev():
            run(tab_r, gidx_r, sidx_r, out_r, dout_r)

    return pl.kernel(body, out_type=out_type, mesh=mesh,
                     scratch_types=scratch,
                     compiler_params=pltpu.CompilerParams(
                         use_tc_tiling_on_sc=False))


_agg_deg = _make_agg(True)
_agg = _make_agg(False)

_BR = 1000


def _tc_body(x_ref, a_ref, deg_ref, ws_ref, wn_ref, b_ref, o_ref):
    inv = 1.0 / jnp.maximum(deg_ref[:, 0:1], 1.0)
    hn = jnp.concatenate([a_ref[p] for p in range(NP)], axis=1) * inv
    acc = jnp.dot(x_ref[...], ws_ref[...], preferred_element_type=jnp.float32)
    acc = acc + jnp.dot(hn, wn_ref[...], preferred_element_type=jnp.float32)
    o_ref[...] = jnp.maximum(acc + b_ref[...], 0.0)


def _sage_dense(x, a, deg, ws, wn, b):
    n = x.shape[0]
    return pl.pallas_call(
        _tc_body,
        grid=(n // _BR,),
        in_specs=[
            pl.BlockSpec((_BR, D), lambda i: (i, 0)),
            pl.BlockSpec((NP, _BR, C), lambda i: (0, i, 0)),
            pl.BlockSpec((_BR, 16), lambda i: (i, 0)),
            pl.BlockSpec((D, D), lambda i: (0, 0)),
            pl.BlockSpec((D, D), lambda i: (0, 0)),
            pl.BlockSpec((1, D), lambda i: (0, 0)),
        ],
        out_specs=pl.BlockSpec((_BR, D), lambda i: (i, 0)),
        out_shape=jax.ShapeDtypeStruct((n, D), jnp.float32),
    )(x, a, deg, ws, wn, b.reshape(1, D))


def _prep(edge_index):
    src = edge_index[0]
    dst = edge_index[1]
    pad0 = jnp.zeros((E_PAD - E,), jnp.int32)
    padd = jnp.full((E_PAD - E,), DUMMY, jnp.int32)
    shp = (NS, CPT, CHUNK)
    gidx_f = jnp.stack(
        [jnp.concatenate([src * NP + p, pad0]).reshape(shp)
         for p in range(NP)])
    sidx_f = jnp.concatenate([dst, padd]).reshape(shp)
    gidx_r = jnp.stack(
        [jnp.concatenate([dst * NP + p, pad0]).reshape(shp)
         for p in range(NP)])
    sidx_r = jnp.concatenate([src, padd]).reshape(shp)
    zeros = jnp.zeros((ROWS_PT, C), jnp.float32)
    zeros16 = jnp.zeros((ROWS_PT, 16), jnp.float32)
    ones = jnp.ones((CHUNK, 16), jnp.float32)
    return gidx_f, sidx_f, gidx_r, sidx_r, zeros, zeros16, ones


def kernel(x_source, x_destination, edge_index, Ws_ship_0, Wn_ship_0,
           b_ship_0, Ws_rev_0, Wn_rev_0, b_rev_0, Ws_ship_1, Wn_ship_1,
           b_ship_1, Ws_rev_1, Wn_rev_1, b_rev_1):
    idx = _prep(edge_index)

    a_d, a_s, deg_d, deg_s = _agg_deg(
        x_source.reshape(NP * N_SRC, C),
        x_destination.reshape(NP * N_DST, C), *idx)
    h_d = _sage_dense(x_destination, a_d, deg_d, Ws_ship_0, Wn_ship_0,
                      b_ship_0)
    h_s = _sage_dense(x_source, a_s, deg_s, Ws_rev_0, Wn_rev_0, b_rev_0)

    a_d1, a_s1 = _agg(h_s.reshape(NP * N_SRC, C),
                      h_d.reshape(NP * N_DST, C), *idx)
    h_d1 = _sage_dense(h_d, a_d1, deg_d, Ws_ship_1, Wn_ship_1, b_ship_1)
    h_s1 = _sage_dense(h_s, a_s1, deg_s, Ws_rev_1, Wn_rev_1, b_rev_1)
    return (h_s1, h_d1)


# speedup vs baseline: 1.0216x; 1.0216x over previous
import jax
import jax.numpy as jnp
from jax import lax
from jax.experimental import pallas as pl
from jax.experimental.pallas import tpu as pltpu
from jax.experimental.pallas import tpu_sc as plsc

N_SRC = 10000
N_DST = 10000
E = 640000
D = 128

NC = 2
NS = 16
NP = 2
C = D // NP
CHUNK = 128
BLK = 8
NBLK = 40
CPT = BLK * NBLK
E_PAD = NS * CPT * CHUNK
ACC_ROWS = 10016
ROWS_PT = ACC_ROWS // NS
DUMMY = 10008


def _make_agg(with_deg: bool):
    mesh = plsc.VectorSubcoreMesh(core_axis_name="c", subcore_axis_name="s")

    out_type = [
        pltpu.HBM((NP, ACC_ROWS, C), jnp.float32),
        pltpu.HBM((NP, ACC_ROWS, C), jnp.float32),
    ]
    if with_deg:
        out_type += [
            pltpu.HBM((ACC_ROWS, 16), jnp.float32),
            pltpu.HBM((ACC_ROWS, 16), jnp.float32),
        ]

    scratch = [
        pltpu.VMEM((BLK, CHUNK), jnp.int32),
        pltpu.VMEM((BLK, CHUNK), jnp.int32),
        pltpu.VMEM((CHUNK, C), jnp.float32),
        pltpu.VMEM((CHUNK, C), jnp.float32),
        pltpu.SemaphoreType.DMA,
        pltpu.SemaphoreType.DMA,
        pltpu.SemaphoreType.DMA,
        pltpu.SemaphoreType.DMA,
        pltpu.VMEM_SHARED((ACC_ROWS, C), jnp.float32),
    ]
    if with_deg:
        scratch += [
            pltpu.VMEM((CHUNK, 16), jnp.float32),
            pltpu.VMEM_SHARED((ACC_ROWS, 16), jnp.float32),
        ]

    def body(tab_f, tab_r, gidx_f, sidx_f, gidx_r, sidx_r, zeros_hbm,
             zeros16_hbm, ones_hbm, *rest):
        if with_deg:
            (out_f, out_r, dout_f, dout_r,
             gidx_v, sidx_v, rows_a, rows_b, gsem_a, gsem_b, ssem_a, ssem_b,
             acc, ones_v, dacc) = rest
        else:
            (out_f, out_r, gidx_v, sidx_v, rows_a, rows_b, gsem_a, gsem_b,
             ssem_a, ssem_b, acc) = rest
            dout_f = dout_r = ones_v = dacc = None

        cid = lax.axis_index("c")
        sid = lax.axis_index("s")
        base = sid * ROWS_PT

        def run(tab, gidx_hbm, sidx_hbm, out_hbm, dout_hbm):
            if with_deg:
                pltpu.sync_copy(ones_hbm, ones_v)
                pltpu.sync_copy(zeros16_hbm, dacc.at[pl.ds(base, ROWS_PT)])

            for p in range(NP):
                pltpu.sync_copy(zeros_hbm, acc.at[pl.ds(base, ROWS_PT)])
                plsc.subcore_barrier()

                deg_here = with_deg and p == 0

                def blk(b, carry):
                    pltpu.sync_copy(gidx_hbm.at[p, sid, pl.ds(b * BLK, BLK)],
                                    gidx_v)
                    pltpu.sync_copy(sidx_hbm.at[sid, pl.ds(b * BLK, BLK)],
                                    sidx_v)

                    bufs = (rows_a, rows_b)
                    gsems = (gsem_a, gsem_b)
                    ssems = (ssem_a, ssem_b)

                    def gather(j, x):
                        return pltpu.async_copy(tab.at[gidx_v.at[j]],
                                                bufs[x], gsems[x])

                    gh = [gather(0, 0), None]
                    sh = [None, None]
                    for j in range(BLK):
                        x = j % 2
                        y = 1 - x
                        if j + 1 < BLK:
                            if sh[y] is not None:
                                sh[y].wait()
                            gh[y] = gather(j + 1, y)
                        gh[x].wait()
                        sh[x] = pltpu.async_copy(
                            bufs[x], acc.at[sidx_v.at[j]], ssems[x],
                            add=True)
                        if deg_here:
                            pltpu.sync_copy(ones_v, dacc.at[sidx_v.at[j]],
                                            add=True)
                    sh[0].wait()
                    sh[1].wait()
                    return carry

                lax.fori_loop(0, NBLK, blk, 0)
                plsc.subcore_barrier()

                pltpu.sync_copy(acc.at[pl.ds(base, ROWS_PT)],
                                out_hbm.at[p, pl.ds(base, ROWS_PT)])
                if with_deg and p == 0:
                    pltpu.sync_copy(dacc.at[pl.ds(base, ROWS_PT)],
                                    dout_hbm.at[pl.ds(base, ROWS_PT)])
                plsc.subcore_barrier()

        @pl.when(cid == 0)
        def _fwd():
            run(tab_f, gidx_f, sidx_f, out_f, dout_f)

        @pl.when(cid == 1)
        def _rev():
            run(tab_r, gidx_r, sidx_r, out_r, dout_r)

    return pl.kernel(body, out_type=out_type, mesh=mesh,
                     scratch_types=scratch,
                     compiler_params=pltpu.CompilerParams(
                         use_tc_tiling_on_sc=False))


_agg_deg = _make_agg(True)
_agg = _make_agg(False)

_BR = 1000


def _tc_body(x_ref, a_ref, deg_ref, ws_ref, wn_ref, b_ref, o_ref):
    inv = 1.0 / jnp.maximum(deg_ref[:, 0:1], 1.0)
    hn = jnp.concatenate([a_ref[p] for p in range(NP)], axis=1) * inv
    acc = jnp.dot(x_ref[...], ws_ref[...], preferred_element_type=jnp.float32)
    acc = acc + jnp.dot(hn, wn_ref[...], preferred_element_type=jnp.float32)
    o_ref[...] = jnp.maximum(acc + b_ref[...], 0.0)


def _sage_dense(x, a, deg, ws, wn, b):
    n = x.shape[0]
    return pl.pallas_call(
        _tc_body,
        grid=(n // _BR,),
        in_specs=[
            pl.BlockSpec((_BR, D), lambda i: (i, 0)),
            pl.BlockSpec((NP, _BR, C), lambda i: (0, i, 0)),
            pl.BlockSpec((_BR, 16), lambda i: (i, 0)),
            pl.BlockSpec((D, D), lambda i: (0, 0)),
            pl.BlockSpec((D, D), lambda i: (0, 0)),
            pl.BlockSpec((1, D), lambda i: (0, 0)),
        ],
        out_specs=pl.BlockSpec((_BR, D), lambda i: (i, 0)),
        out_shape=jax.ShapeDtypeStruct((n, D), jnp.float32),
    )(x, a, deg, ws, wn, b.reshape(1, D))


def _prep(edge_index):
    src = edge_index[0]
    dst = edge_index[1]
    pad0 = jnp.zeros((E_PAD - E,), jnp.int32)
    padd = jnp.full((E_PAD - E,), DUMMY, jnp.int32)
    shp = (NS, CPT, CHUNK)
    gidx_f = jnp.stack(
        [jnp.concatenate([src * NP + p, pad0]).reshape(shp)
         for p in range(NP)])
    sidx_f = jnp.concatenate([dst, padd]).reshape(shp)
    gidx_r = jnp.stack(
        [jnp.concatenate([dst * NP + p, pad0]).reshape(shp)
         for p in range(NP)])
    sidx_r = jnp.concatenate([src, padd]).reshape(shp)
    zeros = jnp.zeros((ROWS_PT, C), jnp.float32)
    zeros16 = jnp.zeros((ROWS_PT, 16), jnp.float32)
    ones = jnp.ones((CHUNK, 16), jnp.float32)
    return gidx_f, sidx_f, gidx_r, sidx_r, zeros, zeros16, ones


def kernel(x_source, x_destination, edge_index, Ws_ship_0, Wn_ship_0,
           b_ship_0, Ws_rev_0, Wn_rev_0, b_rev_0, Ws_ship_1, Wn_ship_1,
           b_ship_1, Ws_rev_1, Wn_rev_1, b_rev_1):
    idx = _prep(edge_index)

    a_d, a_s, deg_d, deg_s = _agg_deg(
        x_source.reshape(NP * N_SRC, C),
        x_destination.reshape(NP * N_DST, C), *idx)
    h_d = _sage_dense(x_destination, a_d, deg_d, Ws_ship_0, Wn_ship_0,
                      b_ship_0)
    h_s = _sage_dense(x_source, a_s, deg_s, Ws_rev_0, Wn_rev_0, b_rev_0)

    a_d1, a_s1 = _agg(h_s.reshape(NP * N_SRC, C),
                      h_d.reshape(NP * N_DST, C), *idx)
    h_d1 = _sage_dense(h_d, a_d1, deg_d, Ws_ship_1, Wn_ship_1, b_ship_1)
    h_s1 = _sage_dense(h_s, a_s1, deg_s, Ws_rev_1, Wn_rev_1, b_rev_1)
    return (h_s1, h_d1)


# speedup vs baseline: 1.0863x; 1.0634x over previous
import jax
import jax.numpy as jnp
from jax import lax
from jax.experimental import pallas as pl
from jax.experimental.pallas import tpu as pltpu
from jax.experimental.pallas import tpu_sc as plsc

N_SRC = 10000
N_DST = 10000
E = 640000
D = 128

NC = 2
NS = 16
CHUNK = 128
BLK = 8
NBLK = 40
CPT = BLK * NBLK
E_PAD = NS * CPT * CHUNK
ACC_ROWS = 10016
ROWS_PT = ACC_ROWS // NS
DUMMY = 10008


def _make_agg():
    mesh = plsc.VectorSubcoreMesh(core_axis_name="c", subcore_axis_name="s")

    out_type = [
        pltpu.HBM((ACC_ROWS, D), jnp.float32),
        pltpu.HBM((ACC_ROWS, D), jnp.float32),
        pltpu.HBM((ACC_ROWS, 16), jnp.float32),
        pltpu.HBM((ACC_ROWS, 16), jnp.float32),
    ]

    scratch = [
        pltpu.VMEM((BLK, CHUNK), jnp.int32),
        pltpu.VMEM((BLK, CHUNK), jnp.int32),
        pltpu.VMEM((CHUNK, D), jnp.float32),
        pltpu.VMEM((CHUNK, D), jnp.float32),
        pltpu.SemaphoreType.DMA,
        pltpu.SemaphoreType.DMA,
        pltpu.SemaphoreType.DMA,
        pltpu.SemaphoreType.DMA,
        pltpu.VMEM_SHARED((ACC_ROWS, D), jnp.float32),
        pltpu.VMEM((CHUNK, 16), jnp.float32),
        pltpu.VMEM_SHARED((ACC_ROWS, 16), jnp.float32),
    ]

    def body(tab_f, tab_r, gidx_f, sidx_f, gidx_r, sidx_r, zeros_hbm,
             zeros16_hbm, ones_hbm,
             out_f, out_r, dout_f, dout_r,
             gidx_v, sidx_v, rows_a, rows_b, gsem_a, gsem_b, ssem_a, ssem_b,
             acc, ones_v, dacc):
        cid = lax.axis_index("c")
        sid = lax.axis_index("s")
        base = sid * ROWS_PT

        def run(tab, gidx_hbm, sidx_hbm, out_hbm, dout_hbm):
            pltpu.sync_copy(ones_hbm, ones_v)
            pltpu.sync_copy(zeros16_hbm, dacc.at[pl.ds(base, ROWS_PT)])
            pltpu.sync_copy(zeros_hbm, acc.at[pl.ds(base, ROWS_PT)])
            plsc.subcore_barrier()

            def blk(b, carry):
                pltpu.sync_copy(gidx_hbm.at[sid, pl.ds(b * BLK, BLK)],
                                gidx_v)
                pltpu.sync_copy(sidx_hbm.at[sid, pl.ds(b * BLK, BLK)],
                                sidx_v)

                bufs = (rows_a, rows_b)
                gsems = (gsem_a, gsem_b)
                ssems = (ssem_a, ssem_b)

                def gather(j, x):
                    return pltpu.async_copy(tab.at[gidx_v.at[j]],
                                            bufs[x], gsems[x])

                gh = [gather(0, 0), None]
                sh = [None, None]
                for j in range(BLK):
                    x = j % 2
                    y = 1 - x
                    if j + 1 < BLK:
                        if sh[y] is not None:
                            sh[y].wait()
                        gh[y] = gather(j + 1, y)
                    gh[x].wait()
                    sh[x] = pltpu.async_copy(
                        bufs[x], acc.at[sidx_v.at[j]], ssems[x],
                        add=True)
                    pltpu.sync_copy(ones_v, dacc.at[sidx_v.at[j]],
                                    add=True)
                sh[0].wait()
                sh[1].wait()
                return carry

            lax.fori_loop(0, NBLK, blk, 0)
            plsc.subcore_barrier()

            pltpu.sync_copy(acc.at[pl.ds(base, ROWS_PT)],
                            out_hbm.at[pl.ds(base, ROWS_PT)])
            pltpu.sync_copy(dacc.at[pl.ds(base, ROWS_PT)],
                            dout_hbm.at[pl.ds(base, ROWS_PT)])
            plsc.subcore_barrier()

        @pl.when(cid == 0)
        def _fwd():
            run(tab_f, gidx_f, sidx_f, out_f, dout_f)

        @pl.when(cid == 1)
        def _rev():
            run(tab_r, gidx_r, sidx_r, out_r, dout_r)

    return pl.kernel(body, out_type=out_type, mesh=mesh,
                     scratch_types=scratch,
                     compiler_params=pltpu.CompilerParams(
                         use_tc_tiling_on_sc=False))


_agg_deg = _make_agg()

_BR = 1000


def _tc_body(x_ref, a_ref, deg_ref, ws_ref, wn_ref, b_ref, o_ref):
    inv = 1.0 / jnp.maximum(deg_ref[:, 0:1], 1.0)
    hn = a_ref[...] * inv
    acc = jnp.dot(x_ref[...], ws_ref[...], preferred_element_type=jnp.float32)
    acc = acc + jnp.dot(hn, wn_ref[...], preferred_element_type=jnp.float32)
    o_ref[...] = jnp.maximum(acc + b_ref[...], 0.0)


def _sage_dense(x, a, deg, ws, wn, b):
    n = x.shape[0]
    return pl.pallas_call(
        _tc_body,
        grid=(n // _BR,),
        in_specs=[
            pl.BlockSpec((_BR, D), lambda i: (i, 0)),
            pl.BlockSpec((_BR, D), lambda i: (i, 0)),
            pl.BlockSpec((_BR, 16), lambda i: (i, 0)),
            pl.BlockSpec((D, D), lambda i: (0, 0)),
            pl.BlockSpec((D, D), lambda i: (0, 0)),
            pl.BlockSpec((1, D), lambda i: (0, 0)),
        ],
        out_specs=pl.BlockSpec((_BR, D), lambda i: (i, 0)),
        out_shape=jax.ShapeDtypeStruct((n, D), jnp.float32),
    )(x, a[:n], deg[:n], ws, wn, b.reshape(1, D))


def _prep(edge_index):
    src = edge_index[0]
    dst = edge_index[1]
    pad0 = jnp.zeros((E_PAD - E,), jnp.int32)
    padd = jnp.full((E_PAD - E,), DUMMY, jnp.int32)
    shp = (NS, CPT, CHUNK)
    gidx_f = jnp.concatenate([src, pad0]).reshape(shp)
    sidx_f = jnp.concatenate([dst, padd]).reshape(shp)
    gidx_r = jnp.concatenate([dst, pad0]).reshape(shp)
    sidx_r = jnp.concatenate([src, padd]).reshape(shp)
    zeros = jnp.zeros((ROWS_PT, D), jnp.float32)
    zeros16 = jnp.zeros((ROWS_PT, 16), jnp.float32)
    ones = jnp.ones((CHUNK, 16), jnp.float32)
    return gidx_f, sidx_f, gidx_r, sidx_r, zeros, zeros16, ones


def kernel(x_source, x_destination, edge_index, Ws_ship_0, Wn_ship_0,
           b_ship_0, Ws_rev_0, Wn_rev_0, b_rev_0, Ws_ship_1, Wn_ship_1,
           b_ship_1, Ws_rev_1, Wn_rev_1, b_rev_1):
    idx = _prep(edge_index)

    a_d, a_s, deg_d, deg_s = _agg_deg(x_source, x_destination, *idx)
    h_d = _sage_dense(x_destination, a_d, deg_d, Ws_ship_0, Wn_ship_0,
                      b_ship_0)
    h_s = _sage_dense(x_source, a_s, deg_s, Ws_rev_0, Wn_rev_0, b_rev_0)

    a_d1, a_s1, _, _ = _agg_deg(h_s, h_d, *idx)
    h_d1 = _sage_dense(h_d, a_d1, deg_d, Ws_ship_1, Wn_ship_1, b_ship_1)
    h_s1 = _sage_dense(h_s, a_s1, deg_s, Ws_rev_1, Wn_rev_1, b_rev_1)
    return (h_s1, h_d1)


# speedup vs baseline: 1.1265x; 1.0370x over previous
import jax
import jax.numpy as jnp
from jax import lax
from jax.experimental import pallas as pl
from jax.experimental.pallas import tpu as pltpu
from jax.experimental.pallas import tpu_sc as plsc

N_SRC = 10000
N_DST = 10000
E = 640000
D = 128

NC = 2
NS = 16
CHUNK = 128
BLK = 16
NBLK = 20
CPT = BLK * NBLK
E_PAD = NS * CPT * CHUNK
ACC_ROWS = 10016
ROWS_PT = ACC_ROWS // NS
DUMMY = 10008


def _make_agg():
    mesh = plsc.VectorSubcoreMesh(core_axis_name="c", subcore_axis_name="s")

    out_type = [
        pltpu.HBM((ACC_ROWS, D), jnp.float32),
        pltpu.HBM((ACC_ROWS, D), jnp.float32),
        pltpu.HBM((ACC_ROWS, 16), jnp.float32),
        pltpu.HBM((ACC_ROWS, 16), jnp.float32),
    ]

    scratch = [
        pltpu.VMEM((BLK, CHUNK), jnp.int32),
        pltpu.VMEM((BLK, CHUNK), jnp.int32),
        pltpu.VMEM((CHUNK, D), jnp.float32),
        pltpu.VMEM((CHUNK, D), jnp.float32),
        pltpu.SemaphoreType.DMA,
        pltpu.SemaphoreType.DMA,
        pltpu.SemaphoreType.DMA,
        pltpu.SemaphoreType.DMA,
        pltpu.SemaphoreType.DMA,
        pltpu.VMEM_SHARED((ACC_ROWS, D), jnp.float32),
        pltpu.VMEM((CHUNK, 16), jnp.float32),
        pltpu.VMEM_SHARED((ACC_ROWS, 16), jnp.float32),
    ]

    def body(tab_f, tab_r, gidx_f, sidx_f, gidx_r, sidx_r, zeros_hbm,
             zeros16_hbm, ones_hbm,
             out_f, out_r, dout_f, dout_r,
             gidx_v, sidx_v, rows_a, rows_b, gsem_a, gsem_b, ssem_a, ssem_b,
             dsem, acc, ones_v, dacc):
        cid = lax.axis_index("c")
        sid = lax.axis_index("s")
        base = sid * ROWS_PT

        def run(tab, gidx_hbm, sidx_hbm, out_hbm, dout_hbm):
            pltpu.sync_copy(ones_hbm, ones_v)
            pltpu.sync_copy(zeros16_hbm, dacc.at[pl.ds(base, ROWS_PT)])
            pltpu.sync_copy(zeros_hbm, acc.at[pl.ds(base, ROWS_PT)])
            plsc.subcore_barrier()

            def blk(b, carry):
                pltpu.sync_copy(gidx_hbm.at[sid, pl.ds(b * BLK, BLK)],
                                gidx_v)
                pltpu.sync_copy(sidx_hbm.at[sid, pl.ds(b * BLK, BLK)],
                                sidx_v)

                bufs = (rows_a, rows_b)
                gsems = (gsem_a, gsem_b)
                ssems = (ssem_a, ssem_b)

                def gather(j, x):
                    return pltpu.async_copy(tab.at[gidx_v.at[j]],
                                            bufs[x], gsems[x])

                gh = [gather(0, 0), None]
                sh = [None, None]
                dh = None
                for j in range(BLK):
                    x = j % 2
                    y = 1 - x
                    if j + 1 < BLK:
                        if sh[y] is not None:
                            sh[y].wait()
                        gh[y] = gather(j + 1, y)
                    gh[x].wait()
                    sh[x] = pltpu.async_copy(
                        bufs[x], acc.at[sidx_v.at[j]], ssems[x],
                        add=True)
                    if dh is not None:
                        dh.wait()
                    dh = pltpu.async_copy(ones_v, dacc.at[sidx_v.at[j]],
                                          dsem, add=True)
                sh[0].wait()
                sh[1].wait()
                dh.wait()
                return carry

            lax.fori_loop(0, NBLK, blk, 0)
            plsc.subcore_barrier()

            pltpu.sync_copy(acc.at[pl.ds(base, ROWS_PT)],
                            out_hbm.at[pl.ds(base, ROWS_PT)])
            pltpu.sync_copy(dacc.at[pl.ds(base, ROWS_PT)],
                            dout_hbm.at[pl.ds(base, ROWS_PT)])
            plsc.subcore_barrier()

        @pl.when(cid == 0)
        def _fwd():
            run(tab_f, gidx_f, sidx_f, out_f, dout_f)

        @pl.when(cid == 1)
        def _rev():
            run(tab_r, gidx_r, sidx_r, out_r, dout_r)

    return pl.kernel(body, out_type=out_type, mesh=mesh,
                     scratch_types=scratch,
                     compiler_params=pltpu.CompilerParams(
                         use_tc_tiling_on_sc=False))


_agg_deg = _make_agg()

_BR = 1000


def _tc_body(x_ref, a_ref, deg_ref, ws_ref, wn_ref, b_ref, o_ref):
    inv = 1.0 / jnp.maximum(deg_ref[:, 0:1], 1.0)
    hn = a_ref[...] * inv
    acc = jnp.dot(x_ref[...], ws_ref[...], preferred_element_type=jnp.float32)
    acc = acc + jnp.dot(hn, wn_ref[...], preferred_element_type=jnp.float32)
    o_ref[...] = jnp.maximum(acc + b_ref[...], 0.0)


def _sage_dense(x, a, deg, ws, wn, b):
    n = x.shape[0]
    return pl.pallas_call(
        _tc_body,
        grid=(n // _BR,),
        in_specs=[
            pl.BlockSpec((_BR, D), lambda i: (i, 0)),
            pl.BlockSpec((_BR, D), lambda i: (i, 0)),
            pl.BlockSpec((_BR, 16), lambda i: (i, 0)),
            pl.BlockSpec((D, D), lambda i: (0, 0)),
            pl.BlockSpec((D, D), lambda i: (0, 0)),
            pl.BlockSpec((1, D), lambda i: (0, 0)),
        ],
        out_specs=pl.BlockSpec((_BR, D), lambda i: (i, 0)),
        out_shape=jax.ShapeDtypeStruct((n, D), jnp.float32),
    )(x, a[:n], deg[:n], ws, wn, b.reshape(1, D))


def _prep(edge_index):
    src = edge_index[0]
    dst = edge_index[1]
    pad0 = jnp.zeros((E_PAD - E,), jnp.int32)
    padd = jnp.full((E_PAD - E,), DUMMY, jnp.int32)
    shp = (NS, CPT, CHUNK)
    gidx_f = jnp.concatenate([src, pad0]).reshape(shp)
    sidx_f = jnp.concatenate([dst, padd]).reshape(shp)
    gidx_r = jnp.concatenate([dst, pad0]).reshape(shp)
    sidx_r = jnp.concatenate([src, padd]).reshape(shp)
    zeros = jnp.zeros((ROWS_PT, D), jnp.float32)
    zeros16 = jnp.zeros((ROWS_PT, 16), jnp.float32)
    ones = jnp.ones((CHUNK, 16), jnp.float32)
    return gidx_f, sidx_f, gidx_r, sidx_r, zeros, zeros16, ones


def kernel(x_source, x_destination, edge_index, Ws_ship_0, Wn_ship_0,
           b_ship_0, Ws_rev_0, Wn_rev_0, b_rev_0, Ws_ship_1, Wn_ship_1,
           b_ship_1, Ws_rev_1, Wn_rev_1, b_rev_1):
    idx = _prep(edge_index)

    a_d, a_s, deg_d, deg_s = _agg_deg(x_source, x_destination, *idx)
    h_d = _sage_dense(x_destination, a_d, deg_d, Ws_ship_0, Wn_ship_0,
                      b_ship_0)
    h_s = _sage_dense(x_source, a_s, deg_s, Ws_rev_0, Wn_rev_0, b_rev_0)

    a_d1, a_s1, _, _ = _agg_deg(h_s, h_d, *idx)
    h_d1 = _sage_dense(h_d, a_d1, deg_d, Ws_ship_1, Wn_ship_1, b_ship_1)
    h_s1 = _sage_dense(h_s, a_s1, deg_s, Ws_rev_1, Wn_rev_1, b_rev_1)
    return (h_s1, h_d1)


# speedup vs baseline: 1.1404x; 1.0123x over previous
import jax
import jax.numpy as jnp
from jax import lax
from jax.experimental import pallas as pl
from jax.experimental.pallas import tpu as pltpu
from jax.experimental.pallas import tpu_sc as plsc

N_SRC = 10000
N_DST = 10000
E = 640000
D = 128

NC = 2
NS = 16
CHUNK = 64
NBUF = 4
BLK = 32
NBLK = 20
CPT = BLK * NBLK
E_PAD = NS * CPT * CHUNK
ACC_ROWS = 10016
ROWS_PT = ACC_ROWS // NS
DUMMY = 10008


def _make_agg():
    mesh = plsc.VectorSubcoreMesh(core_axis_name="c", subcore_axis_name="s")

    out_type = [
        pltpu.HBM((ACC_ROWS, D), jnp.float32),
        pltpu.HBM((ACC_ROWS, D), jnp.float32),
        pltpu.HBM((ACC_ROWS, 16), jnp.float32),
        pltpu.HBM((ACC_ROWS, 16), jnp.float32),
    ]

    scratch = [
        pltpu.VMEM((BLK, CHUNK), jnp.int32),
        pltpu.VMEM((BLK, CHUNK), jnp.int32),
        *[pltpu.VMEM((CHUNK, D), jnp.float32) for _ in range(NBUF)],
        *[pltpu.SemaphoreType.DMA for _ in range(NBUF)],
        *[pltpu.SemaphoreType.DMA for _ in range(NBUF)],
        pltpu.SemaphoreType.DMA,
        pltpu.VMEM_SHARED((ACC_ROWS, D), jnp.float32),
        pltpu.VMEM((CHUNK, 16), jnp.float32),
        pltpu.VMEM_SHARED((ACC_ROWS, 16), jnp.float32),
    ]

    def body(tab_f, tab_r, gidx_f, sidx_f, gidx_r, sidx_r, zeros_hbm,
             zeros16_hbm, ones_hbm,
             out_f, out_r, dout_f, dout_r,
             gidx_v, sidx_v, *rest):
        bufs = rest[:NBUF]
        gsems = rest[NBUF:2 * NBUF]
        ssems = rest[2 * NBUF:3 * NBUF]
        dsem, acc, ones_v, dacc = rest[3 * NBUF:]
        cid = lax.axis_index("c")
        sid = lax.axis_index("s")
        base = sid * ROWS_PT

        def run(tab, gidx_hbm, sidx_hbm, out_hbm, dout_hbm):
            pltpu.sync_copy(ones_hbm, ones_v)
            pltpu.sync_copy(zeros16_hbm, dacc.at[pl.ds(base, ROWS_PT)])
            pltpu.sync_copy(zeros_hbm, acc.at[pl.ds(base, ROWS_PT)])
            plsc.subcore_barrier()

            def blk(b, carry):
                pltpu.sync_copy(gidx_hbm.at[sid, pl.ds(b * BLK, BLK)],
                                gidx_v)
                pltpu.sync_copy(sidx_hbm.at[sid, pl.ds(b * BLK, BLK)],
                                sidx_v)

                def gather(j, x):
                    return pltpu.async_copy(tab.at[gidx_v.at[j]],
                                            bufs[x], gsems[x])

                gh = [None] * NBUF
                sh = [None] * NBUF
                dh = None
                for j in range(BLK + NBUF - 1):
                    if j < BLK:
                        x = j % NBUF
                        if sh[x] is not None:
                            sh[x].wait()
                        gh[x] = gather(j, x)
                    if j >= NBUF - 1:
                        k = j - (NBUF - 1)
                        x = k % NBUF
                        gh[x].wait()
                        sh[x] = pltpu.async_copy(
                            bufs[x], acc.at[sidx_v.at[k]], ssems[x],
                            add=True)
                        if dh is not None:
                            dh.wait()
                        dh = pltpu.async_copy(ones_v, dacc.at[sidx_v.at[k]],
                                              dsem, add=True)
                for x in range(NBUF):
                    if sh[x] is not None:
                        sh[x].wait()
                dh.wait()
                return carry

            lax.fori_loop(0, NBLK, blk, 0)
            plsc.subcore_barrier()

            pltpu.sync_copy(acc.at[pl.ds(base, ROWS_PT)],
                            out_hbm.at[pl.ds(base, ROWS_PT)])
            pltpu.sync_copy(dacc.at[pl.ds(base, ROWS_PT)],
                            dout_hbm.at[pl.ds(base, ROWS_PT)])
            plsc.subcore_barrier()

        @pl.when(cid == 0)
        def _fwd():
            run(tab_f, gidx_f, sidx_f, out_f, dout_f)

        @pl.when(cid == 1)
        def _rev():
            run(tab_r, gidx_r, sidx_r, out_r, dout_r)

    return pl.kernel(body, out_type=out_type, mesh=mesh,
                     scratch_types=scratch,
                     compiler_params=pltpu.CompilerParams(
                         use_tc_tiling_on_sc=False))


_agg_deg = _make_agg()

_BR = 1000


def _tc_body(x_ref, a_ref, deg_ref, ws_ref, wn_ref, b_ref, o_ref):
    inv = 1.0 / jnp.maximum(deg_ref[:, 0:1], 1.0)
    hn = a_ref[...] * inv
    acc = jnp.dot(x_ref[...], ws_ref[...], preferred_element_type=jnp.float32)
    acc = acc + jnp.dot(hn, wn_ref[...], preferred_element_type=jnp.float32)
    o_ref[...] = jnp.maximum(acc + b_ref[...], 0.0)


def _sage_dense(x, a, deg, ws, wn, b):
    n = x.shape[0]
    return pl.pallas_call(
        _tc_body,
        grid=(n // _BR,),
        in_specs=[
            pl.BlockSpec((_BR, D), lambda i: (i, 0)),
            pl.BlockSpec((_BR, D), lambda i: (i, 0)),
            pl.BlockSpec((_BR, 16), lambda i: (i, 0)),
            pl.BlockSpec((D, D), lambda i: (0, 0)),
            pl.BlockSpec((D, D), lambda i: (0, 0)),
            pl.BlockSpec((1, D), lambda i: (0, 0)),
        ],
        out_specs=pl.BlockSpec((_BR, D), lambda i: (i, 0)),
        out_shape=jax.ShapeDtypeStruct((n, D), jnp.float32),
    )(x, a[:n], deg[:n], ws, wn, b.reshape(1, D))


def _prep(edge_index):
    src = edge_index[0]
    dst = edge_index[1]
    pad0 = jnp.zeros((E_PAD - E,), jnp.int32)
    padd = jnp.full((E_PAD - E,), DUMMY, jnp.int32)
    shp = (NS, CPT, CHUNK)
    gidx_f = jnp.concatenate([src, pad0]).reshape(shp)
    sidx_f = jnp.concatenate([dst, padd]).reshape(shp)
    gidx_r = jnp.concatenate([dst, pad0]).reshape(shp)
    sidx_r = jnp.concatenate([src, padd]).reshape(shp)
    zeros = jnp.zeros((ROWS_PT, D), jnp.float32)
    zeros16 = jnp.zeros((ROWS_PT, 16), jnp.float32)
    ones = jnp.ones((CHUNK, 16), jnp.float32)
    return gidx_f, sidx_f, gidx_r, sidx_r, zeros, zeros16, ones


def kernel(x_source, x_destination, edge_index, Ws_ship_0, Wn_ship_0,
           b_ship_0, Ws_rev_0, Wn_rev_0, b_rev_0, Ws_ship_1, Wn_ship_1,
           b_ship_1, Ws_rev_1, Wn_rev_1, b_rev_1):
    idx = _prep(edge_index)

    a_d, a_s, deg_d, deg_s = _agg_deg(x_source, x_destination, *idx)
    h_d = _sage_dense(x_destination, a_d, deg_d, Ws_ship_0, Wn_ship_0,
                      b_ship_0)
    h_s = _sage_dense(x_source, a_s, deg_s, Ws_rev_0, Wn_rev_0, b_rev_0)

    a_d1, a_s1, _, _ = _agg_deg(h_s, h_d, *idx)
    h_d1 = _sage_dense(h_d, a_d1, deg_d, Ws_ship_1, Wn_ship_1, b_ship_1)
    h_s1 = _sage_dense(h_s, a_s1, deg_s, Ws_rev_1, Wn_rev_1, b_rev_1)
    return (h_s1, h_d1)


# speedup vs baseline: 1.9702x; 1.7277x over previous
import jax
import jax.numpy as jnp
from jax import lax
from jax.experimental import pallas as pl
from jax.experimental.pallas import tpu as pltpu
from jax.experimental.pallas import tpu_sc as plsc

N_SRC = 10000
N_DST = 10000
E = 640000
D = 128

NC = 2
NS = 16
NP = 2
C = D // NP
CHUNK = 64
NBUF = 4
BLK = 32
NBLK = 20
CPT = BLK * NBLK
E_PAD = NS * CPT * CHUNK
ACC_ROWS = 10016
ROWS_PT = ACC_ROWS // NS
DUMMY = 10008


def _make_agg():
    mesh = plsc.VectorSubcoreMesh(core_axis_name="c", subcore_axis_name="s")

    out_type = [
        pltpu.HBM((NP, ACC_ROWS, C), jnp.float32),
        pltpu.HBM((NP, ACC_ROWS, C), jnp.float32),
        pltpu.HBM((ACC_ROWS, 16), jnp.float32),
        pltpu.HBM((ACC_ROWS, 16), jnp.float32),
    ]

    scratch = [
        pltpu.VMEM((BLK, CHUNK), jnp.int32),
        pltpu.VMEM((BLK, CHUNK), jnp.int32),
        *[pltpu.VMEM((CHUNK, C), jnp.float32) for _ in range(NBUF)],
        *[pltpu.SemaphoreType.DMA for _ in range(NBUF)],
        *[pltpu.SemaphoreType.DMA for _ in range(NBUF)],
        pltpu.SemaphoreType.DMA,
        pltpu.VMEM_SHARED((ACC_ROWS, C), jnp.float32),
        pltpu.VMEM_SHARED((ACC_ROWS, C), jnp.float32),
        pltpu.VMEM((CHUNK, 16), jnp.float32),
        pltpu.VMEM_SHARED((ACC_ROWS, 16), jnp.float32),
    ]

    def body(tab_f, tab_r, gidx_f, sidx_f, gidx_r, sidx_r, zeros_hbm,
             zeros16_hbm, ones_hbm,
             out_f, out_r, dout_f, dout_r,
             gidx_v, sidx_v, *rest):
        bufs = rest[:NBUF]
        gsems = rest[NBUF:2 * NBUF]
        ssems = rest[2 * NBUF:3 * NBUF]
        dsem, acc, tbuf, ones_v, dacc = rest[3 * NBUF:]
        cid = lax.axis_index("c")
        sid = lax.axis_index("s")
        base = sid * ROWS_PT

        def run(tab, gidx_hbm, sidx_hbm, out_hbm, dout_hbm):
            pltpu.sync_copy(ones_hbm, ones_v)
            pltpu.sync_copy(zeros16_hbm, dacc.at[pl.ds(base, ROWS_PT)])

            for p in range(NP):
                pltpu.sync_copy(
                    tab.at[pl.ds(base, ROWS_PT), pl.ds(p * C, C)],
                    tbuf.at[pl.ds(base, ROWS_PT)])
                pltpu.sync_copy(zeros_hbm, acc.at[pl.ds(base, ROWS_PT)])
                plsc.subcore_barrier()
                _walk(p == 0, gidx_hbm, sidx_hbm)
                plsc.subcore_barrier()

                pltpu.sync_copy(acc.at[pl.ds(base, ROWS_PT)],
                                out_hbm.at[p, pl.ds(base, ROWS_PT)])
                if p == 0:
                    pltpu.sync_copy(dacc.at[pl.ds(base, ROWS_PT)],
                                    dout_hbm.at[pl.ds(base, ROWS_PT)])
                plsc.subcore_barrier()

        def _walk(deg_here, gidx_hbm, sidx_hbm):
            def blk(b, carry):
                pltpu.sync_copy(gidx_hbm.at[sid, pl.ds(b * BLK, BLK)],
                                gidx_v)
                pltpu.sync_copy(sidx_hbm.at[sid, pl.ds(b * BLK, BLK)],
                                sidx_v)

                def gather(j, x):
                    return pltpu.async_copy(tbuf.at[gidx_v.at[j]],
                                            bufs[x], gsems[x])

                gh = [None] * NBUF
                sh = [None] * NBUF
                dh = None
                for j in range(BLK + NBUF - 1):
                    if j < BLK:
                        x = j % NBUF
                        if sh[x] is not None:
                            sh[x].wait()
                        gh[x] = gather(j, x)
                    if j >= NBUF - 1:
                        k = j - (NBUF - 1)
                        x = k % NBUF
                        gh[x].wait()
                        sh[x] = pltpu.async_copy(
                            bufs[x], acc.at[sidx_v.at[k]], ssems[x],
                            add=True)
                        if deg_here:
                            if dh is not None:
                                dh.wait()
                            dh = pltpu.async_copy(
                                ones_v, dacc.at[sidx_v.at[k]],
                                dsem, add=True)
                for x in range(NBUF):
                    if sh[x] is not None:
                        sh[x].wait()
                if dh is not None:
                    dh.wait()
                return carry

            lax.fori_loop(0, NBLK, blk, 0)

        @pl.when(cid == 0)
        def _fwd():
            run(tab_f, gidx_f, sidx_f, out_f, dout_f)

        @pl.when(cid == 1)
        def _rev():
            run(tab_r, gidx_r, sidx_r, out_r, dout_r)

    return pl.kernel(body, out_type=out_type, mesh=mesh,
                     scratch_types=scratch,
                     compiler_params=pltpu.CompilerParams(
                         use_tc_tiling_on_sc=False))


_agg_deg = _make_agg()

_BR = 1000


def _tc_body(x_ref, a_ref, deg_ref, ws_ref, wn_ref, b_ref, o_ref):
    inv = 1.0 / jnp.maximum(deg_ref[:, 0:1], 1.0)
    hn = jnp.concatenate([a_ref[p] for p in range(NP)], axis=1) * inv
    acc = jnp.dot(x_ref[...], ws_ref[...], preferred_element_type=jnp.float32)
    acc = acc + jnp.dot(hn, wn_ref[...], preferred_element_type=jnp.float32)
    o_ref[...] = jnp.maximum(acc + b_ref[...], 0.0)


def _sage_dense(x, a, deg, ws, wn, b):
    n = x.shape[0]
    return pl.pallas_call(
        _tc_body,
        grid=(n // _BR,),
        in_specs=[
            pl.BlockSpec((_BR, D), lambda i: (i, 0)),
            pl.BlockSpec((NP, _BR, C), lambda i: (0, i, 0)),
            pl.BlockSpec((_BR, 16), lambda i: (i, 0)),
            pl.BlockSpec((D, D), lambda i: (0, 0)),
            pl.BlockSpec((D, D), lambda i: (0, 0)),
            pl.BlockSpec((1, D), lambda i: (0, 0)),
        ],
        out_specs=pl.BlockSpec((_BR, D), lambda i: (i, 0)),
        out_shape=jax.ShapeDtypeStruct((n, D), jnp.float32),
    )(x, a, deg, ws, wn, b.reshape(1, D))


def _prep(edge_index):
    src = edge_index[0]
    dst = edge_index[1]
    pad0 = jnp.zeros((E_PAD - E,), jnp.int32)
    padd = jnp.full((E_PAD - E,), DUMMY, jnp.int32)
    shp = (NS, CPT, CHUNK)
    gidx_f = jnp.concatenate([src, pad0]).reshape(shp)
    sidx_f = jnp.concatenate([dst, padd]).reshape(shp)
    gidx_r = jnp.concatenate([dst, pad0]).reshape(shp)
    sidx_r = jnp.concatenate([src, padd]).reshape(shp)
    zeros = jnp.zeros((ROWS_PT, C), jnp.float32)
    zeros16 = jnp.zeros((ROWS_PT, 16), jnp.float32)
    ones = jnp.ones((CHUNK, 16), jnp.float32)
    return gidx_f, sidx_f, gidx_r, sidx_r, zeros, zeros16, ones


def kernel(x_source, x_destination, edge_index, Ws_ship_0, Wn_ship_0,
           b_ship_0, Ws_rev_0, Wn_rev_0, b_rev_0, Ws_ship_1, Wn_ship_1,
           b_ship_1, Ws_rev_1, Wn_rev_1, b_rev_1):
    idx = _prep(edge_index)
    pad = jnp.zeros((ACC_ROWS - N_SRC, D), jnp.float32)

    a_d, a_s, deg_d, deg_s = _agg_deg(
        jnp.concatenate([x_source, pad]),
        jnp.concatenate([x_destination, pad]), *idx)
    h_d = _sage_dense(x_destination, a_d, deg_d, Ws_ship_0, Wn_ship_0,
                      b_ship_0)
    h_s = _sage_dense(x_source, a_s, deg_s, Ws_rev_0, Wn_rev_0, b_rev_0)

    a_d1, a_s1, _, _ = _agg_deg(
        jnp.concatenate([h_s, pad]), jnp.concatenate([h_d, pad]), *idx)
    h_d1 = _sage_dense(h_d, a_d1, deg_d, Ws_ship_1, Wn_ship_1, b_ship_1)
    h_s1 = _sage_dense(h_s, a_s1, deg_s, Ws_rev_1, Wn_rev_1, b_rev_1)
    return (h_s1, h_d1)
